# baseline (speedup 1.0000x reference)
import functools

import jax
import jax.numpy as jnp
from jax import lax
from jax.experimental import pallas as pl
from jax.experimental.pallas import tpu as pltpu
from jax.experimental.pallas import tpu_sc as plsc

N = 10000
E = 320000
F = 128

NC = 2
NS = 16
CHUNK = 128
NBUF = 2
G = 40
K_PER_TILE = 80
NBLK = NC * NS * K_PER_TILE
E_PAD = NBLK * CHUNK
N_PAD = 10112
ROWS_PER_TILE = N_PAD // NS

_mesh = plsc.VectorSubcoreMesh(core_axis_name="c", subcore_axis_name="s")


@functools.partial(
    pl.kernel,
    out_type=jax.ShapeDtypeStruct((NC, N_PAD, F), jnp.float32),
    mesh=_mesh,
    scratch_types=[
        pltpu.VMEM((1, CHUNK), jnp.int32),
        pltpu.VMEM((1, CHUNK), jnp.int32),
        pltpu.VMEM((CHUNK, F), jnp.float32),
        pltpu.VMEM_SHARED((N_PAD, F), jnp.float32),
    ],
)
def _seg_sum_sc(table_hbm, src_hbm, dst_hbm, zeros_hbm, out_hbm,
                src_v, dst_v, rows_v, acc_sh):
    c = lax.axis_index("c")
    s = lax.axis_index("s")
    wid = c * NS + s
    rbase = s * ROWS_PER_TILE
    base = wid * K_PER_TILE
    pltpu.sync_copy(zeros_hbm.at[pl.ds(rbase, ROWS_PER_TILE)],
                    acc_sh.at[pl.ds(rbase, ROWS_PER_TILE)])
    plsc.subcore_barrier()

    @pl.loop(0, K_PER_TILE)
    def _(j):
        pltpu.sync_copy(src_hbm.at[base + j], src_v.at[0])
        pltpu.sync_copy(dst_hbm.at[base + j], dst_v.at[0])
        pltpu.sync_copy(table_hbm.at[src_v.at[0]], rows_v)
        pltpu.sync_copy(rows_v, acc_sh.at[dst_v.at[0]], add=True)

    plsc.subcore_barrier()
    pltpu.sync_copy(acc_sh.at[pl.ds(rbase, ROWS_PER_TILE)],
                    out_hbm.at[c, pl.ds(rbase, ROWS_PER_TILE)])


_BR = 1000


def _row_spec():
    return pl.BlockSpec((_BR, F), lambda i: (i, 0))


def _full_spec():
    return pl.BlockSpec((F, F), lambda i: (0, 0))


def _bias_spec():
    return pl.BlockSpec((1, F), lambda i: (0, 0))


def _dot_t(a, w):
    return lax.dot_general(a, w, (((1,), (1,)), ((), ())),
                           preferred_element_type=jnp.float32)


def _k1_body(x_ref, wl_ref, bl_ref, wr_ref, xproj_ref, xr1_ref):
    x = x_ref[...]
    xproj_ref[...] = _dot_t(x, wl_ref[...]) + bl_ref[...]
    xr1_ref[...] = _dot_t(x, wr_ref[...])


_tc_k1 = pl.pallas_call(
    _k1_body,
    grid=(N // _BR,),
    in_specs=[_row_spec(), _full_spec(), _bias_spec(), _full_spec()],
    out_specs=[_row_spec(), _row_spec()],
    out_shape=[jax.ShapeDtypeStruct((N, F), jnp.float32)] * 2,
)


def _k3_body(a0_ref, a1_ref, xr1_ref, xproj_ref, wrel_ref, brel_ref,
             wroot2_ref, h_ref, hr2_ref):
    agg = a0_ref[...] + a1_ref[...]
    t = _dot_t(agg, wrel_ref[...]) + brel_ref[...] + xr1_ref[...]
    h = jnp.maximum(t, 0.0) + xproj_ref[...]
    h_ref[...] = h
    hr2_ref[...] = _dot_t(h, wroot2_ref[...])


_tc_k3 = pl.pallas_call(
    _k3_body,
    grid=(N // _BR,),
    in_specs=[_row_spec(), _row_spec(), _row_spec(), _row_spec(),
              _full_spec(), _bias_spec(), _full_spec()],
    out_specs=[_row_spec(), _row_spec()],
    out_shape=[jax.ShapeDtypeStruct((N, F), jnp.float32)] * 2,
)


def _k5_body(a0_ref, a1_ref, hr2_ref, xproj_ref, wrel_ref, brel_ref, out_ref):
    agg = a0_ref[...] + a1_ref[...]
    t = _dot_t(agg, wrel_ref[...]) + brel_ref[...] + hr2_ref[...]
    out_ref[...] = jnp.maximum(t, 0.0) + xproj_ref[...]


_tc_k5 = pl.pallas_call(
    _k5_body,
    grid=(N // _BR,),
    in_specs=[_row_spec(), _row_spec(), _row_spec(), _row_spec(),
              _full_spec(), _bias_spec()],
    out_specs=_row_spec(),
    out_shape=jax.ShapeDtypeStruct((N, F), jnp.float32),
)


@jax.jit
def kernel(x, edge_index, W_lin, b_lin, W_rel1, b_rel1, W_root1,
           W_rel2, b_rel2, W_root2):
    ei = edge_index.astype(jnp.int32)
    pad = E_PAD - E
    src_p = jnp.concatenate(
        [ei[0], jnp.zeros((pad,), jnp.int32)]).reshape(NBLK, CHUNK)
    dst_p = jnp.concatenate(
        [ei[1], jnp.full((pad,), N, jnp.int32)]).reshape(NBLK, CHUNK)
    zeros = jnp.zeros((N_PAD, F), jnp.float32)

    bl = b_lin.reshape(1, F)
    br1 = b_rel1.reshape(1, F)
    br2 = b_rel2.reshape(1, F)

    xproj, xr1 = _tc_k1(x, W_lin, bl, W_root1)
    parts1 = _seg_sum_sc(x, src_p, dst_p, zeros)
    h, hr2 = _tc_k3(parts1[0, :N], parts1[1, :N], xr1, xproj,
                    W_rel1, br1, W_root2)
    parts2 = _seg_sum_sc(h, src_p, dst_p, zeros)
    return _tc_k5(parts2[0, :N], parts2[1, :N], hr2, xproj, W_rel2, br2)


# speedup vs baseline: 1.4051x; 1.4051x over previous
import functools

import jax
import jax.numpy as jnp
from jax import lax
from jax.experimental import pallas as pl
from jax.experimental.pallas import tpu as pltpu
from jax.experimental.pallas import tpu_sc as plsc

N = 10000
E = 320000
F = 128

NC = 2
NS = 16
CHUNK = 128
K_PER_TILE = -(-E // (NC * NS * CHUNK))
NBLK = NC * NS * K_PER_TILE
E_PAD = NBLK * CHUNK
N_PAD = 10112
ROWS_PER_TILE = N_PAD // NS

_mesh = plsc.VectorSubcoreMesh(core_axis_name="c", subcore_axis_name="s")


@functools.partial(
    pl.kernel,
    out_type=jax.ShapeDtypeStruct((NC, N_PAD, F), jnp.float32),
    mesh=_mesh,
    scratch_types=[
        pltpu.VMEM((1, CHUNK), jnp.int32),
        pltpu.VMEM((1, CHUNK), jnp.int32),
        pltpu.VMEM((CHUNK, F), jnp.float32),
        pltpu.VMEM_SHARED((N_PAD, F), jnp.float32),
    ],
)
def _seg_sum_sc(table_hbm, src_hbm, dst_hbm, zeros_hbm, out_hbm,
                src_v, dst_v, rows_v, acc_sh):
    c = lax.axis_index("c")
    s = lax.axis_index("s")
    wid = c * NS + s
    rbase = s * ROWS_PER_TILE
    base = wid * K_PER_TILE
    pltpu.sync_copy(zeros_hbm.at[pl.ds(rbase, ROWS_PER_TILE)],
                    acc_sh.at[pl.ds(rbase, ROWS_PER_TILE)])
    plsc.subcore_barrier()

    @pl.loop(0, K_PER_TILE)
    def _(j):
        pltpu.sync_copy(src_hbm.at[base + j], src_v.at[0])
        pltpu.sync_copy(dst_hbm.at[base + j], dst_v.at[0])
        pltpu.sync_copy(table_hbm.at[src_v.at[0]], rows_v)
        pltpu.sync_copy(rows_v, acc_sh.at[dst_v.at[0]], add=True)

    plsc.subcore_barrier()
    pltpu.sync_copy(acc_sh.at[pl.ds(rbase, ROWS_PER_TILE)],
                    out_hbm.at[c, pl.ds(rbase, ROWS_PER_TILE)])


_BR = 1000


def _row_spec():
    return pl.BlockSpec((_BR, F), lambda i: (i, 0))


def _full_spec():
    return pl.BlockSpec((F, F), lambda i: (0, 0))


def _bias_spec():
    return pl.BlockSpec((1, F), lambda i: (0, 0))


def _dot_t(a, w):
    return lax.dot_general(a, w, (((1,), (1,)), ((), ())),
                           preferred_element_type=jnp.float32)


def _k1_body(x_ref, wl_ref, bl_ref, wr_ref, xproj_ref, xr1_ref):
    x = x_ref[...]
    xproj_ref[...] = _dot_t(x, wl_ref[...]) + bl_ref[...]
    xr1_ref[...] = _dot_t(x, wr_ref[...])


_tc_k1 = pl.pallas_call(
    _k1_body,
    grid=(N // _BR,),
    in_specs=[_row_spec(), _full_spec(), _bias_spec(), _full_spec()],
    out_specs=[_row_spec(), _row_spec()],
    out_shape=[jax.ShapeDtypeStruct((N, F), jnp.float32)] * 2,
)


def _k3_body(a0_ref, a1_ref, xr1_ref, xproj_ref, wrel_ref, brel_ref,
             wroot2_ref, h_ref, hr2_ref):
    agg = a0_ref[...] + a1_ref[...]
    t = _dot_t(agg, wrel_ref[...]) + brel_ref[...] + xr1_ref[...]
    h = jnp.maximum(t, 0.0) + xproj_ref[...]
    h_ref[...] = h
    hr2_ref[...] = _dot_t(h, wroot2_ref[...])


_tc_k3 = pl.pallas_call(
    _k3_body,
    grid=(N // _BR,),
    in_specs=[_row_spec(), _row_spec(), _row_spec(), _row_spec(),
              _full_spec(), _bias_spec(), _full_spec()],
    out_specs=[_row_spec(), _row_spec()],
    out_shape=[jax.ShapeDtypeStruct((N, F), jnp.float32)] * 2,
)


def _k5_body(a0_ref, a1_ref, hr2_ref, xproj_ref, wrel_ref, brel_ref, out_ref):
    agg = a0_ref[...] + a1_ref[...]
    t = _dot_t(agg, wrel_ref[...]) + brel_ref[...] + hr2_ref[...]
    out_ref[...] = jnp.maximum(t, 0.0) + xproj_ref[...]


_tc_k5 = pl.pallas_call(
    _k5_body,
    grid=(N // _BR,),
    in_specs=[_row_spec(), _row_spec(), _row_spec(), _row_spec(),
              _full_spec(), _bias_spec()],
    out_specs=_row_spec(),
    out_shape=jax.ShapeDtypeStruct((N, F), jnp.float32),
)


@jax.jit
def kernel(x, edge_index, W_lin, b_lin, W_rel1, b_rel1, W_root1,
           W_rel2, b_rel2, W_root2):
    ei = edge_index.astype(jnp.int32)
    pad = E_PAD - E
    src_p = jnp.concatenate(
        [ei[0], jnp.zeros((pad,), jnp.int32)]).reshape(NBLK, CHUNK)
    dump = N + (jnp.arange(pad, dtype=jnp.int32) % (N_PAD - N))
    dst_p = jnp.concatenate([ei[1], dump]).reshape(NBLK, CHUNK)
    zeros = jnp.zeros((N_PAD, F), jnp.float32)

    bl = b_lin.reshape(1, F)
    br1 = b_rel1.reshape(1, F)
    br2 = b_rel2.reshape(1, F)

    xproj, xr1 = _tc_k1(x, W_lin, bl, W_root1)
    parts1 = _seg_sum_sc(x, src_p, dst_p, zeros)
    h, hr2 = _tc_k3(parts1[0, :N], parts1[1, :N], xr1, xproj,
                    W_rel1, br1, W_root2)
    parts2 = _seg_sum_sc(h, src_p, dst_p, zeros)
    return _tc_k5(parts2[0, :N], parts2[1, :N], hr2, xproj, W_rel2, br2)


# speedup vs baseline: 2.2904x; 1.6301x over previous
import functools

import jax
import jax.numpy as jnp
from jax import lax
from jax.experimental import pallas as pl
from jax.experimental.pallas import tpu as pltpu
from jax.experimental.pallas import tpu_sc as plsc

N = 10000
E = 320000
F = 128

NC = 2
NS = 16
CHUNK = 128
K_PER_TILE = -(-E // (NC * NS * CHUNK))
NBLK = NC * NS * K_PER_TILE
E_PAD = NBLK * CHUNK
N_PAD = 10112
ROWS_PER_TILE = N_PAD // NS

_mesh = plsc.VectorSubcoreMesh(core_axis_name="c", subcore_axis_name="s")


@functools.partial(
    pl.kernel,
    out_type=jax.ShapeDtypeStruct((NC, N_PAD, F), jnp.float32),
    mesh=_mesh,
    scratch_types=[
        pltpu.VMEM((1, CHUNK), jnp.int32),
        pltpu.VMEM((1, CHUNK), jnp.int32),
        pltpu.VMEM((CHUNK, F), jnp.float32),
        pltpu.VMEM_SHARED((N_PAD, F), jnp.float32),
    ],
)
def _seg_sum_sc(table_hbm, src_hbm, dst_hbm, zeros_hbm, out_hbm,
                src_v, dst_v, rows_v, acc_sh):
    c = lax.axis_index("c")
    s = lax.axis_index("s")
    wid = c * NS + s
    rbase = s * ROWS_PER_TILE
    base = wid * K_PER_TILE
    pltpu.sync_copy(zeros_hbm.at[pl.ds(rbase, ROWS_PER_TILE)],
                    acc_sh.at[pl.ds(rbase, ROWS_PER_TILE)])
    plsc.subcore_barrier()

    @pl.loop(0, K_PER_TILE)
    def _(j):
        pltpu.sync_copy(src_hbm.at[base + j], src_v.at[0])
        pltpu.sync_copy(dst_hbm.at[base + j], dst_v.at[0])
        pltpu.sync_copy(table_hbm.at[src_v.at[0]], rows_v)
        pltpu.sync_copy(rows_v, acc_sh.at[dst_v.at[0]], add=True)

    plsc.subcore_barrier()
    pltpu.sync_copy(acc_sh.at[pl.ds(rbase, ROWS_PER_TILE)],
                    out_hbm.at[c, pl.ds(rbase, ROWS_PER_TILE)])


_BR = 1000


def _row_spec():
    return pl.BlockSpec((_BR, F), lambda i: (i, 0))


def _full_spec():
    return pl.BlockSpec((F, F), lambda i: (0, 0))


def _bias_spec():
    return pl.BlockSpec((1, F), lambda i: (0, 0))


def _dot_t(a, w):
    return lax.dot_general(a, w, (((1,), (1,)), ((), ())),
                           preferred_element_type=jnp.float32)


def _k1_body(x_ref, wl_ref, bl_ref, wr_ref, xproj_ref, xr1_ref):
    x = x_ref[...]
    xproj_ref[...] = _dot_t(x, wl_ref[...]) + bl_ref[...]
    xr1_ref[...] = _dot_t(x, wr_ref[...])


_tc_k1 = pl.pallas_call(
    _k1_body,
    grid=(N // _BR,),
    in_specs=[_row_spec(), _full_spec(), _bias_spec(), _full_spec()],
    out_specs=[_row_spec(), _row_spec()],
    out_shape=[jax.ShapeDtypeStruct((N, F), jnp.float32)] * 2,
)


def _k3_body(a0_ref, a1_ref, xr1_ref, xproj_ref, wrel_ref, brel_ref,
             wroot2_ref, h_ref, hr2_ref):
    agg = a0_ref[...] + a1_ref[...]
    t = _dot_t(agg, wrel_ref[...]) + brel_ref[...] + xr1_ref[...]
    h = jnp.maximum(t, 0.0) + xproj_ref[...]
    h_ref[...] = h
    hr2_ref[...] = _dot_t(h, wroot2_ref[...])


_tc_k3 = pl.pallas_call(
    _k3_body,
    grid=(N // _BR,),
    in_specs=[_row_spec(), _row_spec(), _row_spec(), _row_spec(),
              _full_spec(), _bias_spec(), _full_spec()],
    out_specs=[_row_spec(), _row_spec()],
    out_shape=[jax.ShapeDtypeStruct((N, F), jnp.float32)] * 2,
)


def _k5_body(a0_ref, a1_ref, hr2_ref, xproj_ref, wrel_ref, brel_ref, out_ref):
    agg = a0_ref[...] + a1_ref[...]
    t = _dot_t(agg, wrel_ref[...]) + brel_ref[...] + hr2_ref[...]
    out_ref[...] = jnp.maximum(t, 0.0) + xproj_ref[...]


_tc_k5 = pl.pallas_call(
    _k5_body,
    grid=(N // _BR,),
    in_specs=[_row_spec(), _row_spec(), _row_spec(), _row_spec(),
              _full_spec(), _bias_spec()],
    out_specs=_row_spec(),
    out_shape=jax.ShapeDtypeStruct((N, F), jnp.float32),
)


@jax.jit
def kernel(x, edge_index, W_lin, b_lin, W_rel1, b_rel1, W_root1,
           W_rel2, b_rel2, W_root2):
    ei = edge_index.astype(jnp.int32)
    pad = E_PAD - E
    pad_ar = jnp.arange(pad, dtype=jnp.int32)
    src_p = jnp.concatenate(
        [ei[0], pad_ar % N]).reshape(NBLK, CHUNK)
    dst_p = jnp.concatenate(
        [ei[1], N + pad_ar % (N_PAD - N)]).reshape(NBLK, CHUNK)
    zeros = jnp.zeros((N_PAD, F), jnp.float32)

    bl = b_lin.reshape(1, F)
    br1 = b_rel1.reshape(1, F)
    br2 = b_rel2.reshape(1, F)

    xproj, xr1 = _tc_k1(x, W_lin, bl, W_root1)
    parts1 = _seg_sum_sc(x, src_p, dst_p, zeros)
    h, hr2 = _tc_k3(parts1[0, :N], parts1[1, :N], xr1, xproj,
                    W_rel1, br1, W_root2)
    parts2 = _seg_sum_sc(h, src_p, dst_p, zeros)
    return _tc_k5(parts2[0, :N], parts2[1, :N], hr2, xproj, W_rel2, br2)


# speedup vs baseline: 4.3503x; 1.8993x over previous
import functools

import jax
import jax.numpy as jnp
from jax import lax
from jax.experimental import pallas as pl
from jax.experimental.pallas import tpu as pltpu
from jax.experimental.pallas import tpu_sc as plsc

N = 10000
E = 320000
F = 128

NC = 2
NS = 16
CHUNK = 128
NBUF = 2
G = 40
K_PER_TILE = 80
NBLK = NC * NS * K_PER_TILE
E_PAD = NBLK * CHUNK
N_PAD = 10112
ROWS_PER_TILE = N_PAD // NS

_mesh = plsc.VectorSubcoreMesh(core_axis_name="c", subcore_axis_name="s")


@functools.partial(
    pl.kernel,
    out_type=jax.ShapeDtypeStruct((NC, N_PAD, F), jnp.float32),
    mesh=_mesh,
    scratch_types=[
        pltpu.VMEM((G, CHUNK), jnp.int32),
        pltpu.VMEM((G, CHUNK), jnp.int32),
        pltpu.VMEM((CHUNK, F), jnp.float32),
        pltpu.VMEM((CHUNK, F), jnp.float32),
        pltpu.SemaphoreType.DMA,
        pltpu.SemaphoreType.DMA,
        pltpu.VMEM_SHARED((N_PAD, F), jnp.float32),
    ],
)
def _seg_sum_sc(table_hbm, src_hbm, dst_hbm, zeros_hbm, out_hbm,
                src_v, dst_v, b0, b1, s0, s1, acc_sh):
    bufs = (b0, b1)
    sems = (s0, s1)
    c = lax.axis_index("c")
    s = lax.axis_index("s")
    wid = c * NS + s
    rbase = s * ROWS_PER_TILE
    base = wid * K_PER_TILE
    pltpu.sync_copy(zeros_hbm.at[pl.ds(rbase, ROWS_PER_TILE)],
                    acc_sh.at[pl.ds(rbase, ROWS_PER_TILE)])
    plsc.subcore_barrier()

    for grp in range(K_PER_TILE // G):
        gbase = base + grp * G
        pltpu.sync_copy(src_hbm.at[pl.ds(gbase, G)], src_v)
        pltpu.sync_copy(dst_hbm.at[pl.ds(gbase, G)], dst_v)

        for b in range(NBUF):
            pltpu.async_copy(table_hbm.at[src_v.at[b]], bufs[b], sems[b])

        @pl.loop(0, G - NBUF, step=NBUF)
        def _(j):
            for b in range(NBUF):
                g = j + b
                pltpu.make_async_copy(
                    table_hbm.at[src_v.at[g]], bufs[b], sems[b]).wait()
                pltpu.sync_copy(bufs[b], acc_sh.at[dst_v.at[g]], add=True)
                pltpu.async_copy(
                    table_hbm.at[src_v.at[g + NBUF]], bufs[b], sems[b])

        for b in range(NBUF):
            g = G - NBUF + b
            pltpu.make_async_copy(
                table_hbm.at[src_v.at[g]], bufs[b], sems[b]).wait()
            pltpu.sync_copy(bufs[b], acc_sh.at[dst_v.at[g]], add=True)

    plsc.subcore_barrier()
    pltpu.sync_copy(acc_sh.at[pl.ds(rbase, ROWS_PER_TILE)],
                    out_hbm.at[c, pl.ds(rbase, ROWS_PER_TILE)])


_BR = 1000


def _row_spec():
    return pl.BlockSpec((_BR, F), lambda i: (i, 0))


def _full_spec():
    return pl.BlockSpec((F, F), lambda i: (0, 0))


def _bias_spec():
    return pl.BlockSpec((1, F), lambda i: (0, 0))


def _dot_t(a, w):
    return lax.dot_general(a, w, (((1,), (1,)), ((), ())),
                           preferred_element_type=jnp.float32)


def _k1_body(x_ref, wl_ref, bl_ref, wr_ref, xproj_ref, xr1_ref):
    x = x_ref[...]
    xproj_ref[...] = _dot_t(x, wl_ref[...]) + bl_ref[...]
    xr1_ref[...] = _dot_t(x, wr_ref[...])


_tc_k1 = pl.pallas_call(
    _k1_body,
    grid=(N // _BR,),
    in_specs=[_row_spec(), _full_spec(), _bias_spec(), _full_spec()],
    out_specs=[_row_spec(), _row_spec()],
    out_shape=[jax.ShapeDtypeStruct((N, F), jnp.float32)] * 2,
)


def _k3_body(a0_ref, a1_ref, xr1_ref, xproj_ref, wrel_ref, brel_ref,
             wroot2_ref, h_ref, hr2_ref):
    agg = a0_ref[...] + a1_ref[...]
    t = _dot_t(agg, wrel_ref[...]) + brel_ref[...] + xr1_ref[...]
    h = jnp.maximum(t, 0.0) + xproj_ref[...]
    h_ref[...] = h
    hr2_ref[...] = _dot_t(h, wroot2_ref[...])


_tc_k3 = pl.pallas_call(
    _k3_body,
    grid=(N // _BR,),
    in_specs=[_row_spec(), _row_spec(), _row_spec(), _row_spec(),
              _full_spec(), _bias_spec(), _full_spec()],
    out_specs=[_row_spec(), _row_spec()],
    out_shape=[jax.ShapeDtypeStruct((N, F), jnp.float32)] * 2,
)


def _k5_body(a0_ref, a1_ref, hr2_ref, xproj_ref, wrel_ref, brel_ref, out_ref):
    agg = a0_ref[...] + a1_ref[...]
    t = _dot_t(agg, wrel_ref[...]) + brel_ref[...] + hr2_ref[...]
    out_ref[...] = jnp.maximum(t, 0.0) + xproj_ref[...]


_tc_k5 = pl.pallas_call(
    _k5_body,
    grid=(N // _BR,),
    in_specs=[_row_spec(), _row_spec(), _row_spec(), _row_spec(),
              _full_spec(), _bias_spec()],
    out_specs=_row_spec(),
    out_shape=jax.ShapeDtypeStruct((N, F), jnp.float32),
)


@jax.jit
def kernel(x, edge_index, W_lin, b_lin, W_rel1, b_rel1, W_root1,
           W_rel2, b_rel2, W_root2):
    ei = edge_index.astype(jnp.int32)
    pad = E_PAD - E
    pad_ar = jnp.arange(pad, dtype=jnp.int32)
    src_p = jnp.concatenate(
        [ei[0], pad_ar % N]).reshape(NBLK, CHUNK)
    dst_p = jnp.concatenate(
        [ei[1], N + pad_ar % (N_PAD - N)]).reshape(NBLK, CHUNK)
    zeros = jnp.zeros((N_PAD, F), jnp.float32)

    bl = b_lin.reshape(1, F)
    br1 = b_rel1.reshape(1, F)
    br2 = b_rel2.reshape(1, F)

    xproj, xr1 = _tc_k1(x, W_lin, bl, W_root1)
    parts1 = _seg_sum_sc(x, src_p, dst_p, zeros)
    h, hr2 = _tc_k3(parts1[0, :N], parts1[1, :N], xr1, xproj,
                    W_rel1, br1, W_root2)
    parts2 = _seg_sum_sc(h, src_p, dst_p, zeros)
    return _tc_k5(parts2[0, :N], parts2[1, :N], hr2, xproj, W_rel2, br2)
